# Initial kernel scaffold; baseline (speedup 1.0000x reference)
import functools

import jax
import jax.numpy as jnp
from jax import lax
from jax.experimental import pallas as pl
from jax.experimental.pallas import tpu as pltpu
from jax.experimental.pallas import tpu_sc as plsc

_EMBED = 64
_BATCH = 4096
_N_NEG = 50
_NW = 32
_BPW = _BATCH // _NW
_NEG_PW = _BATCH * _N_NEG // _NW
_CHUNK = 128
_NCHUNK = _NEG_PW // _CHUNK


def _body(users_hbm, pos_hbm, pot_hbm, negf_hbm, uemb_hbm, iemb_hbm,
          out_u, out_p, out_t, out_n,
          idx_v, negidx_v, rows_v, sem):
    wid = lax.axis_index("s") * 2 + lax.axis_index("c")
    base = wid * _BPW

    for src, table, out in ((users_hbm, uemb_hbm, out_u),
                            (pos_hbm, iemb_hbm, out_p),
                            (pot_hbm, iemb_hbm, out_t)):
        pltpu.sync_copy(src.at[pl.ds(base, _BPW)], idx_v)
        pltpu.async_copy(table.at[idx_v], rows_v, sem).wait()
        pltpu.sync_copy(rows_v, out.at[pl.ds(base, _BPW)])

    nbase = wid * _NEG_PW
    pltpu.sync_copy(negf_hbm.at[pl.ds(nbase, _NEG_PW)], negidx_v.at[...])

    def chunk(j, carry):
        pltpu.async_copy(iemb_hbm.at[negidx_v.at[j]], rows_v, sem).wait()
        pltpu.sync_copy(rows_v, out_n.at[pl.ds(nbase + j * _CHUNK, _CHUNK)])
        return carry

    lax.fori_loop(0, _NCHUNK, chunk, 0)


@functools.partial(
    pl.kernel,
    mesh=plsc.VectorSubcoreMesh(core_axis_name="c", subcore_axis_name="s"),
    out_type=(
        jax.ShapeDtypeStruct((_BATCH, _EMBED), jnp.float32),
        jax.ShapeDtypeStruct((_BATCH, _EMBED), jnp.float32),
        jax.ShapeDtypeStruct((_BATCH, _EMBED), jnp.float32),
        jax.ShapeDtypeStruct((_BATCH * _N_NEG, _EMBED), jnp.float32),
    ),
    scratch_types=[
        pltpu.VMEM((_BPW,), jnp.int32),
        pltpu.VMEM((_NCHUNK, _CHUNK), jnp.int32),
        pltpu.VMEM((_CHUNK, _EMBED), jnp.float32),
        pltpu.SemaphoreType.DMA,
    ],
)
def _sc_gather(*refs):
    _body(*refs)


def kernel(users, pos_items, pot_items, neg_items, user_emb, item_emb):
    negf = neg_items.reshape(-1)
    out_u, out_p, out_t, out_n = _sc_gather(
        users.astype(jnp.int32), pos_items, pot_items, negf,
        user_emb, item_emb)
    return (out_u, out_p, out_t,
            out_n.reshape(_BATCH, _N_NEG, _EMBED))


# speedup vs baseline: 1.4289x; 1.4289x over previous
import functools

import jax
import jax.numpy as jnp
from jax import lax
from jax.experimental import pallas as pl
from jax.experimental.pallas import tpu as pltpu
from jax.experimental.pallas import tpu_sc as plsc

_EMBED = 64
_BATCH = 4096
_N_NEG = 50
_NW = 32
_BPW = _BATCH // _NW
_NEG_PW = _BATCH * _N_NEG // _NW
_CHUNK = 128
_NCHUNK = _NEG_PW // _CHUNK


def _body(users_hbm, pos_hbm, pot_hbm, negf_hbm, uemb_hbm, iemb_hbm,
          out_u, out_p, out_t, out_n,
          idx_v, negidx_v, rows_v, sem):
    wid = lax.axis_index("s") * 2 + lax.axis_index("c")
    base = wid * _BPW

    for src, table, out in ((users_hbm, uemb_hbm, out_u),
                            (pos_hbm, iemb_hbm, out_p),
                            (pot_hbm, iemb_hbm, out_t)):
        pltpu.sync_copy(src.at[pl.ds(base, _BPW)], idx_v)
        pltpu.async_copy(table.at[idx_v], rows_v, sem).wait()
        pltpu.sync_copy(rows_v, out.at[pl.ds(base, _BPW)])

    nbase = wid * _NEG_PW
    pltpu.sync_copy(negf_hbm.at[pl.ds(nbase, _NEG_PW)], negidx_v)

    def chunk(j, carry):
        pltpu.async_copy(
            iemb_hbm.at[negidx_v.at[pl.ds(j * _CHUNK, _CHUNK)]],
            rows_v, sem).wait()
        pltpu.sync_copy(rows_v, out_n.at[pl.ds(nbase + j * _CHUNK, _CHUNK)])
        return carry

    lax.fori_loop(0, _NCHUNK, chunk, 0)


@functools.partial(
    pl.kernel,
    mesh=plsc.VectorSubcoreMesh(core_axis_name="c", subcore_axis_name="s"),
    out_type=(
        jax.ShapeDtypeStruct((_BATCH, _EMBED), jnp.float32),
        jax.ShapeDtypeStruct((_BATCH, _EMBED), jnp.float32),
        jax.ShapeDtypeStruct((_BATCH, _EMBED), jnp.float32),
        jax.ShapeDtypeStruct((_BATCH * _N_NEG, _EMBED), jnp.float32),
    ),
    scratch_types=[
        pltpu.VMEM((_BPW,), jnp.int32),
        pltpu.VMEM((_NEG_PW,), jnp.int32),
        pltpu.VMEM((_CHUNK, _EMBED), jnp.float32),
        pltpu.SemaphoreType.DMA,
    ],
    compiler_params=pltpu.CompilerParams(use_tc_tiling_on_sc=False),
)
def _sc_gather(*refs):
    _body(*refs)


def kernel(users, pos_items, pot_items, neg_items, user_emb, item_emb):
    negf = neg_items.reshape(-1)
    out_u, out_p, out_t, out_n = _sc_gather(
        users.astype(jnp.int32), pos_items, pot_items, negf,
        user_emb, item_emb)
    return (out_u, out_p, out_t,
            out_n.reshape(_BATCH, _N_NEG, _EMBED))


# speedup vs baseline: 1.4960x; 1.0470x over previous
import functools

import jax
import jax.numpy as jnp
from jax import lax
from jax.experimental import pallas as pl
from jax.experimental.pallas import tpu as pltpu
from jax.experimental.pallas import tpu_sc as plsc

_EMBED = 64
_BATCH = 4096
_N_NEG = 50
_NW = 32
_BPW = _BATCH // _NW
_NEG_PW = _BATCH * _N_NEG // _NW
_CHUNK = 128
_NCHUNK = _NEG_PW // _CHUNK
_NBUF = 5


def _body(users_hbm, pos_hbm, pot_hbm, negf_hbm, uemb_hbm, iemb_hbm,
          out_u, out_p, out_t, out_n,
          sidx_v, negidx_v, srows_v, nrows_v,
          sg0, sg1, sg2, swb, gsems, wsems):
    wid = lax.axis_index("s") * 2 + lax.axis_index("c")
    base = wid * _BPW
    nbase = wid * _NEG_PW
    small = ((users_hbm, uemb_hbm, out_u, sg0),
             (pos_hbm, iemb_hbm, out_p, sg1),
             (pot_hbm, iemb_hbm, out_t, sg2))

    for i, (src, table, _, gsem) in enumerate(small):
        pltpu.sync_copy(src.at[pl.ds(base, _BPW)], sidx_v.at[i])
        pltpu.async_copy(table.at[sidx_v.at[i]], srows_v.at[i], gsem)

    pltpu.sync_copy(negf_hbm.at[pl.ds(nbase, _NEG_PW)], negidx_v)

    def fire_gather(g, b):
        pltpu.async_copy(
            iemb_hbm.at[negidx_v.at[pl.ds(g * _CHUNK, _CHUNK)]],
            nrows_v.at[b], gsems[b])

    def wait_and_writeback(g, b):
        pltpu.make_async_copy(iemb_hbm.at[pl.ds(0, _CHUNK)], nrows_v.at[b],
                              gsems[b]).wait()
        pltpu.async_copy(
            nrows_v.at[b], out_n.at[pl.ds(nbase + g * _CHUNK, _CHUNK)],
            wsems[b])

    for b in range(_NBUF):
        fire_gather(b, b)

    for i, (_, table, out, gsem) in enumerate(small):
        pltpu.make_async_copy(table.at[pl.ds(0, _BPW)], srows_v.at[i],
                              gsem).wait()
        pltpu.async_copy(srows_v.at[i], out.at[pl.ds(base, _BPW)], swb)

    def super_step(k, carry):
        for b in range(_NBUF):
            g = k * _NBUF + b
            wait_and_writeback(g, b)
            pltpu.make_async_copy(nrows_v.at[b],
                                  out_n.at[pl.ds(nbase, _CHUNK)],
                                  wsems[b]).wait()
            fire_gather(g + _NBUF, b)
        return carry

    lax.fori_loop(0, _NCHUNK // _NBUF - 1, super_step, 0)

    for b in range(_NBUF):
        wait_and_writeback(_NCHUNK - _NBUF + b, b)
    for b in range(_NBUF):
        pltpu.make_async_copy(nrows_v.at[b],
                              out_n.at[pl.ds(nbase, _CHUNK)],
                              wsems[b]).wait()
    for i, (_, _, out, _) in enumerate(small):
        pltpu.make_async_copy(srows_v.at[i], out.at[pl.ds(base, _BPW)],
                              swb).wait()


@functools.partial(
    pl.kernel,
    mesh=plsc.VectorSubcoreMesh(core_axis_name="c", subcore_axis_name="s"),
    out_type=(
        jax.ShapeDtypeStruct((_BATCH, _EMBED), jnp.float32),
        jax.ShapeDtypeStruct((_BATCH, _EMBED), jnp.float32),
        jax.ShapeDtypeStruct((_BATCH, _EMBED), jnp.float32),
        jax.ShapeDtypeStruct((_BATCH * _N_NEG, _EMBED), jnp.float32),
    ),
    scratch_types=[
        pltpu.VMEM((3, _BPW), jnp.int32),
        pltpu.VMEM((_NEG_PW,), jnp.int32),
        pltpu.VMEM((3, _BPW, _EMBED), jnp.float32),
        pltpu.VMEM((_NBUF, _CHUNK, _EMBED), jnp.float32),
        pltpu.SemaphoreType.DMA,
        pltpu.SemaphoreType.DMA,
        pltpu.SemaphoreType.DMA,
        pltpu.SemaphoreType.DMA,
        [pltpu.SemaphoreType.DMA] * _NBUF,
        [pltpu.SemaphoreType.DMA] * _NBUF,
    ],
    compiler_params=pltpu.CompilerParams(use_tc_tiling_on_sc=False),
)
def _sc_gather(*refs):
    _body(*refs)


def kernel(users, pos_items, pot_items, neg_items, user_emb, item_emb):
    negf = neg_items.reshape(-1)
    out_u, out_p, out_t, out_n = _sc_gather(
        users.astype(jnp.int32), pos_items, pot_items, negf,
        user_emb, item_emb)
    return (out_u, out_p, out_t,
            out_n.reshape(_BATCH, _N_NEG, _EMBED))
